# baseline (device time: 38963 ns/iter reference)
import jax
import jax.numpy as jnp
from jax import lax
from jax.experimental import pallas as pl
from jax.experimental.pallas import tpu as pltpu

_CHUNKS = (16, 16, 32, 48, 64, 64, 64, 64, 48, 48, 32, 16)


def kernel(partial, resid, gamma):
    M, D = resid.shape
    half = M // 2
    p2 = partial.reshape(M, D)
    g2 = gamma.reshape(1, D)

    assert sum(_CHUNKS) == half
    n_chunks = len(_CHUNKS)
    offs = [sum(_CHUNKS[:c]) for c in range(n_chunks)]

    def body(p_ref, r_ref, g_ref, o_ref, yrecv_ref,
             ysend_sems, yrecv_sems, xsend_sems, xrecv_sems):
        my_x = lax.axis_index("x")
        my_y = lax.axis_index("y")
        row0 = my_x * half
        y_nbr = (my_x, 1 - my_y)
        x_nbr = (1 - my_x, my_y)

        barrier_sem = pltpu.get_barrier_semaphore()
        for nbr in (y_nbr, x_nbr):
            pl.semaphore_signal(
                barrier_sem, inc=1,
                device_id=nbr, device_id_type=pl.DeviceIdType.MESH,
            )
        pl.semaphore_wait(barrier_sem, 2)

        rdmas_y = []
        for c in range(n_chunks):
            rdma_y = pltpu.make_async_remote_copy(
                src_ref=p_ref.at[pl.ds(row0 + offs[c], _CHUNKS[c]), :],
                dst_ref=yrecv_ref.at[pl.ds(offs[c], _CHUNKS[c]), :],
                send_sem=ysend_sems.at[c],
                recv_sem=yrecv_sems.at[c],
                device_id=y_nbr,
                device_id_type=pl.DeviceIdType.MESH,
            )
            rdma_y.start()
            rdmas_y.append(rdma_y)

        rdmas_x = []
        for c in range(n_chunks):
            rdmas_y[c].wait_recv()
            yh = p_ref[pl.ds(row0 + offs[c], _CHUNKS[c]), :] \
                + yrecv_ref[pl.ds(offs[c], _CHUNKS[c]), :] \
                + r_ref[pl.ds(row0 + offs[c], _CHUNKS[c]), :]
            rms = jnp.sqrt(jnp.mean(yh * yh, axis=1, keepdims=True) + 1e-6)
            o_ref[pl.ds(row0 + offs[c], _CHUNKS[c]), :] = yh / rms * g_ref[:, :]

            rdma_x = pltpu.make_async_remote_copy(
                src_ref=o_ref.at[pl.ds(row0 + offs[c], _CHUNKS[c]), :],
                dst_ref=o_ref.at[pl.ds(row0 + offs[c], _CHUNKS[c]), :],
                send_sem=xsend_sems.at[c],
                recv_sem=xrecv_sems.at[c],
                device_id=x_nbr,
                device_id_type=pl.DeviceIdType.MESH,
            )
            rdma_x.start()
            rdmas_x.append(rdma_x)

        for c in range(n_chunks):
            rdmas_y[c].wait_send()
            rdmas_x[c].wait()

    return pl.pallas_call(
        body,
        out_shape=jax.ShapeDtypeStruct((M, D), jnp.float32),
        in_specs=[
            pl.BlockSpec(memory_space=pltpu.VMEM),
            pl.BlockSpec(memory_space=pltpu.VMEM),
            pl.BlockSpec(memory_space=pltpu.VMEM),
        ],
        out_specs=pl.BlockSpec(memory_space=pltpu.VMEM),
        scratch_shapes=[
            pltpu.VMEM((half, D), jnp.float32),
            pltpu.SemaphoreType.DMA((n_chunks,)),
            pltpu.SemaphoreType.DMA((n_chunks,)),
            pltpu.SemaphoreType.DMA((n_chunks,)),
            pltpu.SemaphoreType.DMA((n_chunks,)),
        ],
        compiler_params=pltpu.CompilerParams(collective_id=0),
    )(p2, resid, g2)


# device time: 33926 ns/iter; 1.1485x vs baseline; 1.1485x over previous
import jax
import jax.numpy as jnp
from jax import lax
from jax.experimental import pallas as pl
from jax.experimental.pallas import tpu as pltpu


def kernel(partial, resid, gamma):
    M, D = resid.shape
    half = M // 2
    p2 = partial.reshape(M, D)
    g2 = gamma.reshape(1, D)

    n_chunks = 16
    rows = half // n_chunks

    def body(p_ref, r_ref, g_ref, o_ref, pbf, ybf, obf, xbf,
             ysend_sems, yrecv_sems, xsend_sems, xrecv_sems):
        my_x = lax.axis_index("x")
        my_y = lax.axis_index("y")
        row0 = my_x * half
        other0 = (1 - my_x) * half
        y_nbr = (my_x, 1 - my_y)
        x_nbr = (1 - my_x, my_y)

        barrier_sem = pltpu.get_barrier_semaphore()
        for nbr in (y_nbr, x_nbr):
            pl.semaphore_signal(
                barrier_sem, inc=1,
                device_id=nbr, device_id_type=pl.DeviceIdType.MESH,
            )
        pl.semaphore_wait(barrier_sem, 2)

        rdmas_y = []
        for c in range(n_chunks):
            sl = pl.ds(c * rows, rows)
            pbf[sl, :] = p_ref[pl.ds(row0 + c * rows, rows), :].astype(
                jnp.bfloat16)
            rdma_y = pltpu.make_async_remote_copy(
                src_ref=pbf.at[sl, :],
                dst_ref=ybf.at[sl, :],
                send_sem=ysend_sems.at[c],
                recv_sem=yrecv_sems.at[c],
                device_id=y_nbr,
                device_id_type=pl.DeviceIdType.MESH,
            )
            rdma_y.start()
            rdmas_y.append(rdma_y)

        rdmas_x = []
        for c in range(n_chunks):
            rdmas_y[c].wait_recv()
            sl = pl.ds(c * rows, rows)
            yh = p_ref[pl.ds(row0 + c * rows, rows), :] \
                + ybf[sl, :].astype(jnp.float32) \
                + r_ref[pl.ds(row0 + c * rows, rows), :]
            rms = jnp.sqrt(jnp.mean(yh * yh, axis=1, keepdims=True) + 1e-6)
            out = yh / rms * g_ref[:, :]
            o_ref[pl.ds(row0 + c * rows, rows), :] = out
            obf[sl, :] = out.astype(jnp.bfloat16)

            rdma_x = pltpu.make_async_remote_copy(
                src_ref=obf.at[sl, :],
                dst_ref=xbf.at[sl, :],
                send_sem=xsend_sems.at[c],
                recv_sem=xrecv_sems.at[c],
                device_id=x_nbr,
                device_id_type=pl.DeviceIdType.MESH,
            )
            rdma_x.start()
            rdmas_x.append(rdma_x)

            if c >= 1:
                rdmas_x[c - 1].wait_recv()
                psl = pl.ds((c - 1) * rows, rows)
                o_ref[pl.ds(other0 + (c - 1) * rows, rows), :] = \
                    xbf[psl, :].astype(jnp.float32)

        rdmas_x[n_chunks - 1].wait_recv()
        lsl = pl.ds((n_chunks - 1) * rows, rows)
        o_ref[pl.ds(other0 + (n_chunks - 1) * rows, rows), :] = \
            xbf[lsl, :].astype(jnp.float32)

        for c in range(n_chunks):
            rdmas_y[c].wait_send()
            rdmas_x[c].wait_send()

    return pl.pallas_call(
        body,
        out_shape=jax.ShapeDtypeStruct((M, D), jnp.float32),
        in_specs=[
            pl.BlockSpec(memory_space=pltpu.VMEM),
            pl.BlockSpec(memory_space=pltpu.VMEM),
            pl.BlockSpec(memory_space=pltpu.VMEM),
        ],
        out_specs=pl.BlockSpec(memory_space=pltpu.VMEM),
        scratch_shapes=[
            pltpu.VMEM((half, D), jnp.bfloat16),
            pltpu.VMEM((half, D), jnp.bfloat16),
            pltpu.VMEM((half, D), jnp.bfloat16),
            pltpu.VMEM((half, D), jnp.bfloat16),
            pltpu.SemaphoreType.DMA((n_chunks,)),
            pltpu.SemaphoreType.DMA((n_chunks,)),
            pltpu.SemaphoreType.DMA((n_chunks,)),
            pltpu.SemaphoreType.DMA((n_chunks,)),
        ],
        compiler_params=pltpu.CompilerParams(collective_id=0),
    )(p2, resid, g2)


# device time: 25767 ns/iter; 1.5121x vs baseline; 1.3166x over previous
import jax
import jax.numpy as jnp
from jax import lax
from jax.experimental import pallas as pl
from jax.experimental.pallas import tpu as pltpu


def kernel(partial, resid, gamma):
    M, D = resid.shape
    half = M // 2
    p2 = partial.reshape(M, D)
    g2 = gamma.reshape(1, D)

    n_chunks = 16
    rows = half // n_chunks
    lag = 4

    def body(p_ref, r_ref, g_ref, o_ref, pbf, ybf, obf, xbf,
             ysend_sems, yrecv_sems, xsend_sems, xrecv_sems):
        my_x = lax.axis_index("x")
        my_y = lax.axis_index("y")
        row0 = my_x * half
        other0 = (1 - my_x) * half
        y_nbr = (my_x, 1 - my_y)
        x_nbr = (1 - my_x, my_y)

        barrier_sem = pltpu.get_barrier_semaphore()
        for nbr in (y_nbr, x_nbr):
            pl.semaphore_signal(
                barrier_sem, inc=1,
                device_id=nbr, device_id_type=pl.DeviceIdType.MESH,
            )
        pl.semaphore_wait(barrier_sem, 2)

        rdmas_y = []
        for c in range(n_chunks):
            sl = pl.ds(c * rows, rows)
            pbf[sl, :] = p_ref[pl.ds(row0 + c * rows, rows), :].astype(
                jnp.bfloat16)
            rdma_y = pltpu.make_async_remote_copy(
                src_ref=pbf.at[sl, :],
                dst_ref=ybf.at[sl, :],
                send_sem=ysend_sems.at[c],
                recv_sem=yrecv_sems.at[c],
                device_id=y_nbr,
                device_id_type=pl.DeviceIdType.MESH,
            )
            rdma_y.start()
            rdmas_y.append(rdma_y)

        rdmas_x = []
        for c in range(n_chunks):
            rdmas_y[c].wait_recv()
            sl = pl.ds(c * rows, rows)
            yh = p_ref[pl.ds(row0 + c * rows, rows), :] \
                + ybf[sl, :].astype(jnp.float32) \
                + r_ref[pl.ds(row0 + c * rows, rows), :]
            rms = jnp.sqrt(jnp.mean(yh * yh, axis=1, keepdims=True) + 1e-6)
            out = yh / rms * g_ref[:, :]
            o_ref[pl.ds(row0 + c * rows, rows), :] = out
            obf[sl, :] = out.astype(jnp.bfloat16)

            rdma_x = pltpu.make_async_remote_copy(
                src_ref=obf.at[sl, :],
                dst_ref=xbf.at[sl, :],
                send_sem=xsend_sems.at[c],
                recv_sem=xrecv_sems.at[c],
                device_id=x_nbr,
                device_id_type=pl.DeviceIdType.MESH,
            )
            rdma_x.start()
            rdmas_x.append(rdma_x)

            if c >= lag:
                rdmas_x[c - lag].wait_recv()
                psl = pl.ds((c - lag) * rows, rows)
                o_ref[pl.ds(other0 + (c - lag) * rows, rows), :] = \
                    xbf[psl, :].astype(jnp.float32)

        for c in range(max(n_chunks - lag, 0), n_chunks):
            rdmas_x[c].wait_recv()
            lsl = pl.ds(c * rows, rows)
            o_ref[pl.ds(other0 + c * rows, rows), :] = \
                xbf[lsl, :].astype(jnp.float32)

        for c in range(n_chunks):
            rdmas_y[c].wait_send()
            rdmas_x[c].wait_send()

    return pl.pallas_call(
        body,
        out_shape=jax.ShapeDtypeStruct((M, D), jnp.float32),
        in_specs=[
            pl.BlockSpec(memory_space=pltpu.VMEM),
            pl.BlockSpec(memory_space=pltpu.VMEM),
            pl.BlockSpec(memory_space=pltpu.VMEM),
        ],
        out_specs=pl.BlockSpec(memory_space=pltpu.VMEM),
        scratch_shapes=[
            pltpu.VMEM((half, D), jnp.bfloat16),
            pltpu.VMEM((half, D), jnp.bfloat16),
            pltpu.VMEM((half, D), jnp.bfloat16),
            pltpu.VMEM((half, D), jnp.bfloat16),
            pltpu.SemaphoreType.DMA((n_chunks,)),
            pltpu.SemaphoreType.DMA((n_chunks,)),
            pltpu.SemaphoreType.DMA((n_chunks,)),
            pltpu.SemaphoreType.DMA((n_chunks,)),
        ],
        compiler_params=pltpu.CompilerParams(collective_id=0),
    )(p2, resid, g2)
